# baseline (device time: 92315 ns/iter reference)
import jax
import jax.numpy as jnp
from jax import lax
from jax.experimental import pallas as pl
from jax.experimental.pallas import tpu as pltpu

C = 8


def kernel(Q, K, V):
    b, s, h, d = Q.shape
    scale = d ** -0.5
    half = s // 2
    ck = half // C
    nbh = b * h

    Qt = jnp.transpose(Q, (0, 2, 1, 3))
    Kt = jnp.transpose(K, (0, 2, 1, 3))
    Vt = jnp.transpose(V, (0, 2, 1, 3))

    def body(q_ref, k_ref, v_ref, o_ref, d_buf, r_buf, l_ref,
             y_send, y_recv, x_send, x_recv):
        my_x = lax.axis_index("x")
        my_y = lax.axis_index("y")
        my_z = lax.axis_index("z")
        partner = (my_x, 1 - my_y, my_z)
        xnbr = (1 - my_x, my_y, my_z)

        barrier_sem = pltpu.get_barrier_semaphore()
        for nb in (partner, xnbr):
            pl.semaphore_signal(
                barrier_sem, inc=1,
                device_id=nb, device_id_type=pl.DeviceIdType.MESH,
            )
        pl.semaphore_wait(barrier_sem, 2)

        y_rdmas = []
        for c in range(C):
            for t, src in enumerate((k_ref, v_ref)):
                r = pltpu.make_async_remote_copy(
                    src_ref=src.at[:, :, pl.ds(my_x * half + c * ck, ck), :],
                    dst_ref=d_buf.at[t, :, :, pl.ds(c * ck, ck), :],
                    send_sem=y_send.at[2 * c + t],
                    recv_sem=y_recv.at[2 * c + t],
                    device_id=partner, device_id_type=pl.DeviceIdType.MESH,
                )
                r.start()
                y_rdmas.append(r)

        def accum(i, get_k, get_v, init=False, final=False):
            bi = i // h
            hi = i - bi * h
            q = q_ref[bi, hi] * scale
            kk = get_k(bi, hi)
            vv = get_v(bi, hi)
            sc = lax.dot_general(
                q, kk, (((1,), (1,)), ((), ())),
                preferred_element_type=jnp.float32,
            )
            p = jnp.exp(sc)
            lsum = jnp.sum(p, axis=1, keepdims=True)
            o = lax.dot_general(
                p, vv, (((1,), (0,)), ((), ())),
                preferred_element_type=jnp.float32,
            )
            if init:
                l_ref[bi, hi] = lsum
                o_ref[bi, hi] = o
            elif final:
                o_ref[bi, hi] = (o_ref[bi, hi] + o) / (l_ref[bi, hi] + lsum)
            else:
                l_ref[bi, hi] = l_ref[bi, hi] + lsum
                o_ref[bi, hi] = o_ref[bi, hi] + o
            return 0

        lax.fori_loop(
            0, nbh,
            lambda i, _: accum(
                i, lambda bi, hi: k_ref[bi, hi], lambda bi, hi: v_ref[bi, hi],
                init=True,
            ),
            0,
        )

        x_rdmas = []
        for c in range(C):
            y_rdmas[2 * c + 0].wait_recv()
            y_rdmas[2 * c + 1].wait_recv()
            for t in range(2):
                r = pltpu.make_async_remote_copy(
                    src_ref=d_buf.at[t, :, :, pl.ds(c * ck, ck), :],
                    dst_ref=r_buf.at[t, :, :, pl.ds(c * ck, ck), :],
                    send_sem=x_send.at[2 * c + t],
                    recv_sem=x_recv.at[2 * c + t],
                    device_id=xnbr, device_id_type=pl.DeviceIdType.MESH,
                )
                r.start()
                x_rdmas.append(r)

        lax.fori_loop(
            0, nbh,
            lambda i, _: accum(
                i,
                lambda bi, hi: d_buf[0, bi, hi],
                lambda bi, hi: d_buf[1, bi, hi],
                init=False,
            ),
            0,
        )

        for r in x_rdmas:
            r.wait_recv()
        lax.fori_loop(
            0, nbh,
            lambda i, _: accum(
                i,
                lambda bi, hi: r_buf[0, bi, hi],
                lambda bi, hi: r_buf[1, bi, hi],
                final=True,
            ),
            0,
        )

        for r in y_rdmas + x_rdmas:
            r.wait_send()

    out = pl.pallas_call(
        body,
        out_shape=jax.ShapeDtypeStruct((b, h, s, d), jnp.float32),
        in_specs=[pl.BlockSpec(memory_space=pltpu.VMEM)] * 3,
        out_specs=pl.BlockSpec(memory_space=pltpu.VMEM),
        scratch_shapes=[
            pltpu.VMEM((2, b, h, half, d), jnp.float32),
            pltpu.VMEM((2, b, h, half, d), jnp.float32),
            pltpu.VMEM((b, h, s, 1), jnp.float32),
            pltpu.SemaphoreType.DMA((2 * C,)),
            pltpu.SemaphoreType.DMA((2 * C,)),
            pltpu.SemaphoreType.DMA((2 * C,)),
            pltpu.SemaphoreType.DMA((2 * C,)),
        ],
        compiler_params=pltpu.CompilerParams(
            collective_id=0, vmem_limit_bytes=100 * 1024 * 1024
        ),
    )(Qt, Kt, Vt)
    return jnp.transpose(out, (0, 2, 1, 3))


# device time: 87921 ns/iter; 1.0500x vs baseline; 1.0500x over previous
import jax
import jax.numpy as jnp
from jax import lax
from jax.experimental import pallas as pl
from jax.experimental.pallas import tpu as pltpu

C = 8


def kernel(Q, K, V):
    b, s, h, d = Q.shape
    scale = d ** -0.5
    half = s // 2
    ck = half // C
    nbh = b * h

    Qt = jnp.transpose(Q, (0, 2, 1, 3))
    Kt = jnp.transpose(K, (0, 2, 1, 3))
    Vt = jnp.transpose(V, (0, 2, 1, 3))

    def body(q_ref, k_ref, v_ref, o_ref, d_buf, r_buf, l_ref,
             y_send, y_recv, x_send, x_recv):
        my_x = lax.axis_index("x")
        my_y = lax.axis_index("y")
        my_z = lax.axis_index("z")
        partner = (my_x, 1 - my_y, my_z)
        xnbr = (1 - my_x, my_y, my_z)

        barrier_sem = pltpu.get_barrier_semaphore()
        for nb in (partner, xnbr):
            pl.semaphore_signal(
                barrier_sem, inc=1,
                device_id=nb, device_id_type=pl.DeviceIdType.MESH,
            )
        pl.semaphore_wait(barrier_sem, 2)

        y_rdmas = []
        for c in range(C):
            for t, src in enumerate((k_ref, v_ref)):
                r = pltpu.make_async_remote_copy(
                    src_ref=src.at[:, :, pl.ds(my_x * half + c * ck, ck), :],
                    dst_ref=d_buf.at[t, :, :, pl.ds(c * ck, ck), :],
                    send_sem=y_send.at[2 * c + t],
                    recv_sem=y_recv.at[2 * c + t],
                    device_id=partner, device_id_type=pl.DeviceIdType.MESH,
                )
                r.start()
                y_rdmas.append(r)

        def accum(i, get_k, get_v, init=False, final=False):
            bi = i // h
            hi = i - bi * h
            q = q_ref[bi, hi] * scale
            kk = get_k(bi, hi)
            vv = get_v(bi, hi)
            sc = lax.dot_general(
                q, kk, (((1,), (1,)), ((), ())),
                preferred_element_type=jnp.float32,
            )
            p = jnp.exp(sc)
            lsum = jnp.sum(p, axis=1, keepdims=True)
            o = lax.dot_general(
                p, vv, (((1,), (0,)), ((), ())),
                preferred_element_type=jnp.float32,
            )
            if init:
                l_ref[bi, hi] = lsum
                o_ref[bi, hi] = o
            elif final:
                o_ref[bi, hi] = (o_ref[bi, hi] + o) / (l_ref[bi, hi] + lsum)
            else:
                l_ref[bi, hi] = l_ref[bi, hi] + lsum
                o_ref[bi, hi] = o_ref[bi, hi] + o
            return 0

        lax.fori_loop(
            0, nbh,
            lambda i, _: accum(
                i, lambda bi, hi: k_ref[bi, hi], lambda bi, hi: v_ref[bi, hi],
                init=True,
            ),
            0,
        )

        sub = half // 2

        def accum_range(buf, off, n, **kw):
            lax.fori_loop(
                0, nbh,
                lambda i, _: accum(
                    i,
                    lambda bi, hi: buf[0, bi, hi, pl.ds(off, n), :],
                    lambda bi, hi: buf[1, bi, hi, pl.ds(off, n), :],
                    **kw,
                ),
                0,
            )

        x_rdmas = []
        for c in range(C):
            y_rdmas[2 * c + 0].wait_recv()
            y_rdmas[2 * c + 1].wait_recv()
            for t in range(2):
                r = pltpu.make_async_remote_copy(
                    src_ref=d_buf.at[t, :, :, pl.ds(c * ck, ck), :],
                    dst_ref=r_buf.at[t, :, :, pl.ds(c * ck, ck), :],
                    send_sem=x_send.at[2 * c + t],
                    recv_sem=x_recv.at[2 * c + t],
                    device_id=xnbr, device_id_type=pl.DeviceIdType.MESH,
                )
                r.start()
                x_rdmas.append(r)
            if c == C // 2 - 1:
                accum_range(d_buf, 0, sub)
        accum_range(d_buf, sub, sub)

        for c in range(C // 2):
            x_rdmas[2 * c + 0].wait_recv()
            x_rdmas[2 * c + 1].wait_recv()
        accum_range(r_buf, 0, sub)
        for c in range(C // 2, C):
            x_rdmas[2 * c + 0].wait_recv()
            x_rdmas[2 * c + 1].wait_recv()
        accum_range(r_buf, sub, sub, final=True)

        for r in y_rdmas + x_rdmas:
            r.wait_send()

    out = pl.pallas_call(
        body,
        out_shape=jax.ShapeDtypeStruct((b, h, s, d), jnp.float32),
        in_specs=[pl.BlockSpec(memory_space=pltpu.VMEM)] * 3,
        out_specs=pl.BlockSpec(memory_space=pltpu.VMEM),
        scratch_shapes=[
            pltpu.VMEM((2, b, h, half, d), jnp.float32),
            pltpu.VMEM((2, b, h, half, d), jnp.float32),
            pltpu.VMEM((b, h, s, 1), jnp.float32),
            pltpu.SemaphoreType.DMA((2 * C,)),
            pltpu.SemaphoreType.DMA((2 * C,)),
            pltpu.SemaphoreType.DMA((2 * C,)),
            pltpu.SemaphoreType.DMA((2 * C,)),
        ],
        compiler_params=pltpu.CompilerParams(
            collective_id=0, vmem_limit_bytes=100 * 1024 * 1024
        ),
    )(Qt, Kt, Vt)
    return jnp.transpose(out, (0, 2, 1, 3))


# device time: 67204 ns/iter; 1.3737x vs baseline; 1.3083x over previous
import jax
import jax.numpy as jnp
from jax import lax
from jax.experimental import pallas as pl
from jax.experimental.pallas import tpu as pltpu

C = 8


def kernel(Q, K, V):
    b, s, h, d = Q.shape
    scale = d ** -0.5
    half = s // 2
    ck = half // C
    nbh = b * h

    Qt = jnp.transpose(Q, (0, 2, 1, 3))
    Kt = jnp.transpose(K, (0, 2, 1, 3))
    Vt = jnp.transpose(V, (0, 2, 1, 3))

    def body(q_ref, k_ref, v_ref, o_ref, kv_bf, d_buf, r_buf, l_ref,
             y_send, y_recv, x_send, x_recv):
        my_x = lax.axis_index("x")
        my_y = lax.axis_index("y")
        my_z = lax.axis_index("z")
        partner = (my_x, 1 - my_y, my_z)
        xnbr = (1 - my_x, my_y, my_z)

        barrier_sem = pltpu.get_barrier_semaphore()
        for nb in (partner, xnbr):
            pl.semaphore_signal(
                barrier_sem, inc=1,
                device_id=nb, device_id_type=pl.DeviceIdType.MESH,
            )
        pl.semaphore_wait(barrier_sem, 2)

        kv_bf[0] = k_ref[:, :, pl.ds(my_x * half, half), :].astype(jnp.bfloat16)
        kv_bf[1] = v_ref[:, :, pl.ds(my_x * half, half), :].astype(jnp.bfloat16)

        y_rdmas = []
        for c in range(C):
            for t in range(2):
                r = pltpu.make_async_remote_copy(
                    src_ref=kv_bf.at[t, :, :, pl.ds(c * ck, ck), :],
                    dst_ref=d_buf.at[t, :, :, pl.ds(c * ck, ck), :],
                    send_sem=y_send.at[2 * c + t],
                    recv_sem=y_recv.at[2 * c + t],
                    device_id=partner, device_id_type=pl.DeviceIdType.MESH,
                )
                r.start()
                y_rdmas.append(r)

        def accum(i, get_k, get_v, init=False, final=False):
            bi = i // h
            hi = i - bi * h
            q = q_ref[bi, hi] * scale
            kk = get_k(bi, hi)
            vv = get_v(bi, hi)
            sc = lax.dot_general(
                q.astype(kk.dtype), kk, (((1,), (1,)), ((), ())),
                preferred_element_type=jnp.float32,
            )
            p = jnp.exp(sc)
            lsum = jnp.sum(p, axis=1, keepdims=True)
            o = lax.dot_general(
                p.astype(vv.dtype), vv, (((1,), (0,)), ((), ())),
                preferred_element_type=jnp.float32,
            )
            if init:
                l_ref[bi, hi] = lsum
                o_ref[bi, hi] = o
            elif final:
                o_ref[bi, hi] = (o_ref[bi, hi] + o) / (l_ref[bi, hi] + lsum)
            else:
                l_ref[bi, hi] = l_ref[bi, hi] + lsum
                o_ref[bi, hi] = o_ref[bi, hi] + o
            return 0

        lax.fori_loop(
            0, nbh,
            lambda i, _: accum(
                i, lambda bi, hi: k_ref[bi, hi], lambda bi, hi: v_ref[bi, hi],
                init=True,
            ),
            0,
        )

        sub = half // 2

        def accum_range(buf, off, n, **kw):
            lax.fori_loop(
                0, nbh,
                lambda i, _: accum(
                    i,
                    lambda bi, hi: buf[0, bi, hi, pl.ds(off, n), :],
                    lambda bi, hi: buf[1, bi, hi, pl.ds(off, n), :],
                    **kw,
                ),
                0,
            )

        x_rdmas = []
        for c in range(C):
            y_rdmas[2 * c + 0].wait_recv()
            y_rdmas[2 * c + 1].wait_recv()
            for t in range(2):
                r = pltpu.make_async_remote_copy(
                    src_ref=d_buf.at[t, :, :, pl.ds(c * ck, ck), :],
                    dst_ref=r_buf.at[t, :, :, pl.ds(c * ck, ck), :],
                    send_sem=x_send.at[2 * c + t],
                    recv_sem=x_recv.at[2 * c + t],
                    device_id=xnbr, device_id_type=pl.DeviceIdType.MESH,
                )
                r.start()
                x_rdmas.append(r)
            if c == C // 2 - 1:
                accum_range(d_buf, 0, sub)
        accum_range(d_buf, sub, sub)

        for c in range(C // 2):
            x_rdmas[2 * c + 0].wait_recv()
            x_rdmas[2 * c + 1].wait_recv()
        accum_range(r_buf, 0, sub)
        for c in range(C // 2, C):
            x_rdmas[2 * c + 0].wait_recv()
            x_rdmas[2 * c + 1].wait_recv()
        accum_range(r_buf, sub, sub, final=True)

        for r in y_rdmas + x_rdmas:
            r.wait_send()

    out = pl.pallas_call(
        body,
        out_shape=jax.ShapeDtypeStruct((b, h, s, d), jnp.float32),
        in_specs=[pl.BlockSpec(memory_space=pltpu.VMEM)] * 3,
        out_specs=pl.BlockSpec(memory_space=pltpu.VMEM),
        scratch_shapes=[
            pltpu.VMEM((2, b, h, half, d), jnp.bfloat16),
            pltpu.VMEM((2, b, h, half, d), jnp.bfloat16),
            pltpu.VMEM((2, b, h, half, d), jnp.bfloat16),
            pltpu.VMEM((b, h, s, 1), jnp.float32),
            pltpu.SemaphoreType.DMA((2 * C,)),
            pltpu.SemaphoreType.DMA((2 * C,)),
            pltpu.SemaphoreType.DMA((2 * C,)),
            pltpu.SemaphoreType.DMA((2 * C,)),
        ],
        compiler_params=pltpu.CompilerParams(
            collective_id=0, vmem_limit_bytes=100 * 1024 * 1024
        ),
    )(Qt, Kt, Vt)
    return jnp.transpose(out, (0, 2, 1, 3))


# device time: 56417 ns/iter; 1.6363x vs baseline; 1.1912x over previous
import jax
import jax.numpy as jnp
from jax import lax
from jax.experimental import pallas as pl
from jax.experimental.pallas import tpu as pltpu

C = 8


def kernel(Q, K, V):
    b, s, h, d = Q.shape
    scale = d ** -0.5
    half = s // 2
    ck = half // C
    nbh = b * h
    bf = jnp.bfloat16

    Qt = jnp.transpose(Q, (0, 2, 1, 3))
    Kt = jnp.transpose(K, (0, 2, 1, 3))
    Vt = jnp.transpose(V, (0, 2, 1, 3))

    def body(q_ref, k_ref, v_ref, o_ref, kv_bf, d_buf, r_buf, l_ref,
             y_send, y_recv, x_send, x_recv):
        my_x = lax.axis_index("x")
        my_y = lax.axis_index("y")
        my_z = lax.axis_index("z")
        partner = (my_x, 1 - my_y, my_z)
        xnbr = (1 - my_x, my_y, my_z)

        barrier_sem = pltpu.get_barrier_semaphore()
        for nb in (partner, xnbr):
            pl.semaphore_signal(
                barrier_sem, inc=1,
                device_id=nb, device_id_type=pl.DeviceIdType.MESH,
            )
        pl.semaphore_wait(barrier_sem, 2)

        kv_bf[0] = k_ref[:, :, pl.ds(my_x * half, half), :].astype(bf)
        kv_bf[1] = v_ref[:, :, pl.ds(my_x * half, half), :].astype(bf)

        y_rdmas = []
        for c in range(C):
            for t in range(2):
                r = pltpu.make_async_remote_copy(
                    src_ref=kv_bf.at[t, :, :, pl.ds(c * ck, ck), :],
                    dst_ref=d_buf.at[t, :, :, pl.ds(c * ck, ck), :],
                    send_sem=y_send.at[2 * c + t],
                    recv_sem=y_recv.at[2 * c + t],
                    device_id=partner, device_id_type=pl.DeviceIdType.MESH,
                )
                r.start()
                y_rdmas.append(r)

        qs = jnp.reshape((q_ref[...] * scale).astype(bf), (nbh, s, d))
        dn = (((2,), (2,)), ((0,), (0,)))
        dnv = (((2,), (1,)), ((0,), (0,)))

        def accum(kk, vv, init=False, final=False):
            nk = kk.shape[2]
            kk = jnp.reshape(kk, (nbh, nk, d))
            vv = jnp.reshape(vv, (nbh, nk, d))
            sc = lax.dot_general(qs, kk, dn, preferred_element_type=jnp.float32)
            p = jnp.exp(sc)
            lsum = jnp.reshape(jnp.sum(p, axis=2, keepdims=True), (b, h, s, 1))
            o = jnp.reshape(
                lax.dot_general(
                    p.astype(bf), vv, dnv, preferred_element_type=jnp.float32
                ),
                (b, h, s, d),
            )
            if init:
                l_ref[...] = lsum
                o_ref[...] = o
            elif final:
                o_ref[...] = (o_ref[...] + o) / (l_ref[...] + lsum)
            else:
                l_ref[...] = l_ref[...] + lsum
                o_ref[...] = o_ref[...] + o

        accum(k_ref[...].astype(bf), v_ref[...].astype(bf), init=True)

        sub = half // 2
        x_rdmas = []
        for c in range(C):
            y_rdmas[2 * c + 0].wait_recv()
            y_rdmas[2 * c + 1].wait_recv()
            for t in range(2):
                r = pltpu.make_async_remote_copy(
                    src_ref=d_buf.at[t, :, :, pl.ds(c * ck, ck), :],
                    dst_ref=r_buf.at[t, :, :, pl.ds(c * ck, ck), :],
                    send_sem=x_send.at[2 * c + t],
                    recv_sem=x_recv.at[2 * c + t],
                    device_id=xnbr, device_id_type=pl.DeviceIdType.MESH,
                )
                r.start()
                x_rdmas.append(r)
            if c == C // 2 - 1:
                accum(d_buf[0, :, :, pl.ds(0, sub), :],
                      d_buf[1, :, :, pl.ds(0, sub), :])
        accum(d_buf[0, :, :, pl.ds(sub, sub), :],
              d_buf[1, :, :, pl.ds(sub, sub), :])

        for c in range(C // 2):
            x_rdmas[2 * c + 0].wait_recv()
            x_rdmas[2 * c + 1].wait_recv()
        accum(r_buf[0, :, :, pl.ds(0, sub), :],
              r_buf[1, :, :, pl.ds(0, sub), :])
        for c in range(C // 2, C):
            x_rdmas[2 * c + 0].wait_recv()
            x_rdmas[2 * c + 1].wait_recv()
        accum(r_buf[0, :, :, pl.ds(sub, sub), :],
              r_buf[1, :, :, pl.ds(sub, sub), :], final=True)

        for r in y_rdmas + x_rdmas:
            r.wait_send()

    out = pl.pallas_call(
        body,
        out_shape=jax.ShapeDtypeStruct((b, h, s, d), jnp.float32),
        in_specs=[pl.BlockSpec(memory_space=pltpu.VMEM)] * 3,
        out_specs=pl.BlockSpec(memory_space=pltpu.VMEM),
        scratch_shapes=[
            pltpu.VMEM((2, b, h, half, d), bf),
            pltpu.VMEM((2, b, h, half, d), bf),
            pltpu.VMEM((2, b, h, half, d), bf),
            pltpu.VMEM((b, h, s, 1), jnp.float32),
            pltpu.SemaphoreType.DMA((2 * C,)),
            pltpu.SemaphoreType.DMA((2 * C,)),
            pltpu.SemaphoreType.DMA((2 * C,)),
            pltpu.SemaphoreType.DMA((2 * C,)),
        ],
        compiler_params=pltpu.CompilerParams(
            collective_id=0, vmem_limit_bytes=100 * 1024 * 1024
        ),
    )(Qt, Kt, Vt)
    return jnp.transpose(out, (0, 2, 1, 3))


# device time: 56326 ns/iter; 1.6389x vs baseline; 1.0016x over previous
import jax
import jax.numpy as jnp
from jax import lax
from jax.experimental import pallas as pl
from jax.experimental.pallas import tpu as pltpu

C = 8


def kernel(Q, K, V):
    b, s, h, d = Q.shape
    scale = d ** -0.5
    half = s // 2
    ck = half // C
    nbh = b * h
    bf = jnp.bfloat16

    Qt = jnp.transpose(Q, (0, 2, 1, 3))
    Kt = jnp.transpose(K, (0, 2, 1, 3))
    Vt = jnp.transpose(V, (0, 2, 1, 3))

    def body(q_ref, k_ref, v_ref, o_ref, kv_bf, d_buf, r_buf, l_ref,
             y_send, y_recv, x_send, x_recv):
        my_x = lax.axis_index("x")
        my_y = lax.axis_index("y")
        my_z = lax.axis_index("z")
        partner = (my_x, 1 - my_y, my_z)
        xnbr = (1 - my_x, my_y, my_z)

        barrier_sem = pltpu.get_barrier_semaphore()
        for nb in (partner, xnbr):
            pl.semaphore_signal(
                barrier_sem, inc=1,
                device_id=nb, device_id_type=pl.DeviceIdType.MESH,
            )
        pl.semaphore_wait(barrier_sem, 2)

        y_rdmas = []
        for c in range(C):
            for t, src in enumerate((k_ref, v_ref)):
                kv_bf[t, :, :, pl.ds(c * ck, ck), :] = src[
                    :, :, pl.ds(my_x * half + c * ck, ck), :
                ].astype(bf)
                r = pltpu.make_async_remote_copy(
                    src_ref=kv_bf.at[t, :, :, pl.ds(c * ck, ck), :],
                    dst_ref=d_buf.at[t, :, :, pl.ds(c * ck, ck), :],
                    send_sem=y_send.at[2 * c + t],
                    recv_sem=y_recv.at[2 * c + t],
                    device_id=partner, device_id_type=pl.DeviceIdType.MESH,
                )
                r.start()
                y_rdmas.append(r)

        qs = jnp.reshape((q_ref[...] * scale).astype(bf), (nbh, s, d))
        dn = (((2,), (2,)), ((0,), (0,)))
        dnv = (((2,), (1,)), ((0,), (0,)))

        def accum(kk, vv, init=False, final=False):
            nk = kk.shape[2]
            kk = jnp.reshape(kk, (nbh, nk, d))
            vv = jnp.reshape(vv, (nbh, nk, d))
            sc = lax.dot_general(qs, kk, dn, preferred_element_type=jnp.float32)
            p = jnp.exp(sc)
            lsum = jnp.reshape(jnp.sum(p, axis=2, keepdims=True), (b, h, s, 1))
            o = jnp.reshape(
                lax.dot_general(
                    p.astype(bf), vv, dnv, preferred_element_type=jnp.float32
                ),
                (b, h, s, d),
            )
            if init:
                l_ref[...] = lsum
                o_ref[...] = o
            elif final:
                o_ref[...] = (o_ref[...] + o) / (l_ref[...] + lsum)
            else:
                l_ref[...] = l_ref[...] + lsum
                o_ref[...] = o_ref[...] + o

        accum(k_ref[...].astype(bf), v_ref[...].astype(bf), init=True)

        sub = half // 2
        x_rdmas = []
        for c in range(C):
            y_rdmas[2 * c + 0].wait_recv()
            y_rdmas[2 * c + 1].wait_recv()
            for t in range(2):
                r = pltpu.make_async_remote_copy(
                    src_ref=d_buf.at[t, :, :, pl.ds(c * ck, ck), :],
                    dst_ref=r_buf.at[t, :, :, pl.ds(c * ck, ck), :],
                    send_sem=x_send.at[2 * c + t],
                    recv_sem=x_recv.at[2 * c + t],
                    device_id=xnbr, device_id_type=pl.DeviceIdType.MESH,
                )
                r.start()
                x_rdmas.append(r)
            if c == C // 2 - 1:
                accum(d_buf[0, :, :, pl.ds(0, sub), :],
                      d_buf[1, :, :, pl.ds(0, sub), :])
        accum(d_buf[0, :, :, pl.ds(sub, sub), :],
              d_buf[1, :, :, pl.ds(sub, sub), :])

        for c in range(C // 2):
            x_rdmas[2 * c + 0].wait_recv()
            x_rdmas[2 * c + 1].wait_recv()
        accum(r_buf[0, :, :, pl.ds(0, sub), :],
              r_buf[1, :, :, pl.ds(0, sub), :])
        for c in range(C // 2, C):
            x_rdmas[2 * c + 0].wait_recv()
            x_rdmas[2 * c + 1].wait_recv()
        accum(r_buf[0, :, :, pl.ds(sub, sub), :],
              r_buf[1, :, :, pl.ds(sub, sub), :], final=True)

        for r in y_rdmas + x_rdmas:
            r.wait_send()

    out = pl.pallas_call(
        body,
        out_shape=jax.ShapeDtypeStruct((b, h, s, d), jnp.float32),
        in_specs=[pl.BlockSpec(memory_space=pltpu.VMEM)] * 3,
        out_specs=pl.BlockSpec(memory_space=pltpu.VMEM),
        scratch_shapes=[
            pltpu.VMEM((2, b, h, half, d), bf),
            pltpu.VMEM((2, b, h, half, d), bf),
            pltpu.VMEM((2, b, h, half, d), bf),
            pltpu.VMEM((b, h, s, 1), jnp.float32),
            pltpu.SemaphoreType.DMA((2 * C,)),
            pltpu.SemaphoreType.DMA((2 * C,)),
            pltpu.SemaphoreType.DMA((2 * C,)),
            pltpu.SemaphoreType.DMA((2 * C,)),
        ],
        compiler_params=pltpu.CompilerParams(
            collective_id=0, vmem_limit_bytes=100 * 1024 * 1024
        ),
    )(Qt, Kt, Vt)
    return jnp.transpose(out, (0, 2, 1, 3))
